# baseline (speedup 1.0000x reference)
import functools

import jax
import jax.numpy as jnp
from jax import lax
from jax.experimental import pallas as pl
from jax.experimental.pallas import tpu as pltpu
from jax.experimental.pallas import tpu_sc as plsc

N = 10000
D = 128
NPAD = 10240
EPAD = 327680
NW = 32
EPT = EPAD // NW
WIN = 128
NWIN = EPT // WIN
RPT = NPAD // 16

_f32 = jnp.float32
_i32 = jnp.int32


def _mesh():
    return plsc.VectorSubcoreMesh(core_axis_name="c", subcore_axis_name="s",
                                  num_cores=2, num_subcores=16)


def _make_sc_deg_cnt():
    @functools.partial(
        pl.kernel,
        out_type=[
            jax.ShapeDtypeStruct((2, NPAD), _f32),
            jax.ShapeDtypeStruct((2, NPAD), _f32),
        ],
        mesh=_mesh(),
        scratch_types=[
            pltpu.VMEM((NWIN, WIN), _i32),
            pltpu.VMEM((NWIN, WIN), _f32),
            pltpu.VMEM((4, WIN), _i32),
            pltpu.VMEM((WIN,), _f32),
            pltpu.VMEM((RPT,), _f32),
            pltpu.VMEM_SHARED((NPAD,), _f32),
            pltpu.VMEM_SHARED((NPAD,), _f32),
        ] + [pltpu.SemaphoreType.DMA] * 8,
        compiler_params=pltpu.CompilerParams(needs_layout_passes=False),
    )
    def sc_deg_cnt(pk_hbm, w_hbm, deg_out, cnt_out,
                   pkv, wv, didx, onesv, bncv, deg_s, cnt_s, *sems):
        sems_a, sems_b = sems[0:4], sems[4:8]
        c = lax.axis_index("c")
        s = lax.axis_index("s")
        wid = c * 16 + s
        pltpu.sync_copy(pk_hbm.at[pl.ds(wid * NWIN, NWIN), :], pkv)
        pltpu.sync_copy(w_hbm.at[pl.ds(wid * NWIN, NWIN), :], wv)
        zeros16 = jnp.zeros((16,), _f32)
        ones16 = jnp.ones((16,), _f32)
        for k in range(WIN // 16):
            onesv[pl.ds(k * 16, 16)] = ones16

        def zbody(i, carry):
            bncv[pl.ds(i * 16, 16)] = zeros16
            return carry
        lax.fori_loop(0, RPT // 16, zbody, 0)
        pltpu.sync_copy(bncv, deg_s.at[pl.ds(s * RPT, RPT)])
        pltpu.sync_copy(bncv, cnt_s.at[pl.ds(s * RPT, RPT)])
        plsc.subcore_barrier()

        def issue(j, r):
            for k in range(WIN // 16):
                v = pkv[j, pl.ds(k * 16, 16)]
                didx[r, pl.ds(k * 16, 16)] = v >> 16
            pltpu.async_copy(wv.at[j], deg_s.at[didx.at[r]], sems_a[r],
                             add=True)
            pltpu.async_copy(onesv, cnt_s.at[didx.at[r]], sems_b[r],
                             add=True)

        def drain(r):
            pltpu.make_async_copy(wv.at[0], deg_s.at[didx.at[r]],
                                  sems_a[r]).wait()
            pltpu.make_async_copy(onesv, cnt_s.at[didx.at[r]],
                                  sems_b[r]).wait()

        for r in range(4):
            issue(jnp.int32(r), r)

        def body(q, carry):
            for r in range(4):
                drain(r)
                issue(4 * q + r, r)
            return carry
        lax.fori_loop(1, NWIN // 4, body, 0)
        for r in range(4):
            drain(r)
        plsc.subcore_barrier()
        pltpu.sync_copy(deg_s.at[pl.ds(s * RPT, RPT)], bncv)
        pltpu.sync_copy(bncv, deg_out.at[c, pl.ds(s * RPT, RPT)])
        pltpu.sync_copy(cnt_s.at[pl.ds(s * RPT, RPT)], bncv)
        pltpu.sync_copy(bncv, cnt_out.at[c, pl.ds(s * RPT, RPT)])

    return sc_deg_cnt


_WIN = 64
_NWINS = EPT // _WIN
_NROW = EPT // 128
_NBUF = 4


def _make_sc_agg(scaled: bool):
    scratch = [
        pltpu.VMEM((_NROW, 128), _i32),
        pltpu.VMEM((_NBUF, 128), _i32),
        pltpu.VMEM((_NBUF, 128), _i32),
        pltpu.VMEM((_WIN, D), _f32),
        pltpu.VMEM((_WIN, D), _f32),
        pltpu.VMEM((_WIN, D), _f32),
        pltpu.VMEM((_WIN, D), _f32),
        pltpu.VMEM_SHARED((NPAD, D), _f32),
    ] + [pltpu.SemaphoreType.DMA] * (3 * _NBUF if scaled else 2 * _NBUF)
    if scaled:
        scratch.insert(3, pltpu.VMEM((_NBUF, 128), _f32))

    @functools.partial(
        pl.kernel,
        out_type=jax.ShapeDtypeStruct((2, NPAD, D), _f32),
        mesh=_mesh(),
        scratch_types=scratch,
        compiler_params=pltpu.CompilerParams(needs_layout_passes=False),
    )
    def sc_agg(*refs):
        if scaled:
            (pk_hbm, w_hbm, rows_hbm, out_hbm,
             pkv, sidxv, didxv, wwin, gb0, gb1, gb2, gb3, acc_s,
             *sems) = refs
            gsems, ssems, wsems = sems[0:4], sems[4:8], sems[8:12]
        else:
            (pk_hbm, rows_hbm, out_hbm,
             pkv, sidxv, didxv, gb0, gb1, gb2, gb3, acc_s, *sems) = refs
            gsems, ssems = sems[0:4], sems[4:8]
        gbs = [gb0, gb1, gb2, gb3]
        c = lax.axis_index("c")
        s = lax.axis_index("s")
        wid = c * 16 + s
        pltpu.sync_copy(pk_hbm.at[pl.ds(wid * _NROW, _NROW), :], pkv)
        zeros16 = jnp.zeros((16,), _f32)

        def zrow(i, carry):
            for k in range(D // 16):
                gb0[i, pl.ds(k * 16, 16)] = zeros16
            return carry
        lax.fori_loop(0, _WIN, zrow, 0)
        for b in range(RPT // _WIN):
            pltpu.sync_copy(gb0, acc_s.at[pl.ds(s * RPT + b * _WIN, _WIN), :])
        plsc.subcore_barrier()

        mask16 = jnp.full((16,), 0xFFFF, _i32)

        def unpack(wj, h, slot):
            row = wj // 2
            for k in range(_WIN // 16):
                v = pkv[row, pl.ds(h * _WIN + k * 16, 16)]
                sidxv[slot, pl.ds(k * 16, 16)] = v & mask16
                didxv[slot, pl.ds(k * 16, 16)] = v >> 16

        def gather(slot, gsem):
            pltpu.async_copy(
                rows_hbm.at[sidxv.at[slot, pl.ds(0, _WIN)]], gbs[slot],
                gsems[slot] if gsem is None else gsem)

        def gwait(slot):
            pltpu.make_async_copy(
                rows_hbm.at[sidxv.at[slot, pl.ds(0, _WIN)]], gbs[slot],
                gsems[slot]).wait()

        def scatter(slot):
            pltpu.async_copy(gbs[slot],
                             acc_s.at[didxv.at[slot, pl.ds(0, _WIN)]],
                             ssems[slot], add=True)

        def swait(slot):
            pltpu.make_async_copy(gbs[slot],
                                  acc_s.at[didxv.at[slot, pl.ds(0, _WIN)]],
                                  ssems[slot]).wait()

        def wcopy(wj, h, slot):
            pltpu.async_copy(w_hbm.at[wid * _NROW + wj // 2,
                                      pl.ds(h * _WIN, _WIN)],
                             wwin.at[slot, pl.ds(0, _WIN)], wsems[slot])

        def wwait(slot):
            pltpu.make_async_copy(w_hbm.at[wid * _NROW,
                                           pl.ds(0, _WIN)],
                                  wwin.at[slot, pl.ds(0, _WIN)],
                                  wsems[slot]).wait()

        def scale(slot):
            gb = gbs[slot]
            for kk in range(_WIN // 16):
                w16 = wwin[slot, pl.ds(kk * 16, 16)]
                for r in range(16):
                    i = kk * 16 + r
                    sc16 = jnp.broadcast_to(w16[r], (16,))
                    for k in range(D // 16):
                        gb[i, pl.ds(k * 16, 16)] = (
                            gb[i, pl.ds(k * 16, 16)] * sc16)

        def step(j, r, first, wsync):
            gwait(r)
            if scaled:
                if not wsync:
                    wwait(r)
                scale(r)
            scatter(r)
            pm = (r + 2) % _NBUF
            if not first:
                swait(pm)
            nj = lax.rem(j + 2, _NWINS)
            nh = (r + 2) % 2
            unpack(nj, nh, pm)
            if scaled:
                wcopy(nj, nh, pm)
            gather(pm, None)

        for wj in range(2):
            unpack(jnp.int32(wj), wj % 2, wj)
            if scaled:
                pltpu.sync_copy(
                    w_hbm.at[wid * _NROW + wj // 2,
                             pl.ds((wj % 2) * _WIN, _WIN)],
                    wwin.at[wj, pl.ds(0, _WIN)])
            gather(wj, None)
        step(jnp.int32(0), 0, True, True)
        step(jnp.int32(1), 1, True, True)
        step(jnp.int32(2), 2, False, False)
        step(jnp.int32(3), 3, False, False)

        def body(q, carry):
            j = 4 * q
            step(j, 0, False, False)
            step(j + 1, 1, False, False)
            step(j + 2, 2, False, False)
            step(j + 3, 3, False, False)
            return carry
        lax.fori_loop(1, _NWINS // 4, body, 0)
        swait(2)
        swait(3)
        for slot in range(2):
            gwait(slot)
            if scaled:
                wwait(slot)

        plsc.subcore_barrier()
        for b in range(RPT // _WIN):
            pltpu.sync_copy(acc_s.at[pl.ds(s * RPT + b * _WIN, _WIN), :], gb0)
            pltpu.sync_copy(gb0, out_hbm.at[c, pl.ds(s * RPT + b * _WIN, _WIN), :])

    return sc_agg


_BLK = 1000


def _p0(i):
    return (0, i, 0)


def _p1(i):
    return (1, i, 0)


def _tc_matmul_bias(x, W, b_row):
    def body(x_ref, w_ref, b_ref, o_ref):
        o_ref[...] = (jnp.dot(x_ref[...], w_ref[...],
                              preferred_element_type=_f32) + b_ref[...])
    return pl.pallas_call(
        body,
        out_shape=jax.ShapeDtypeStruct((N, D), _f32),
        grid=(N // _BLK,),
        in_specs=[
            pl.BlockSpec((_BLK, D), lambda i: (i, 0)),
            pl.BlockSpec((D, D), lambda i: (0, 0)),
            pl.BlockSpec((1, D), lambda i: (0, 0)),
        ],
        out_specs=pl.BlockSpec((_BLK, D), lambda i: (i, 0)),
    )(x, W, b_row)


def _tc_norm1(h, deg_p, cnt_p):
    def body(h_ref, d0, d1, c0, c1, ht_ref, dinv_ref, invc_ref):
        deg = d0[...].reshape(_BLK, 1) + d1[...].reshape(_BLK, 1)
        dinv = lax.rsqrt(jnp.clip(deg, 1e-12, None))
        cnt = jnp.clip(c0[...].reshape(_BLK, 1) + c1[...].reshape(_BLK, 1),
                       1.0, None)
        dinv_ref[...] = dinv
        invc_ref[...] = 1.0 / cnt
        ht_ref[...] = h_ref[...] * dinv
    part = [pl.BlockSpec((1, _BLK, 1), _p0), pl.BlockSpec((1, _BLK, 1), _p1)]
    col = pl.BlockSpec((_BLK, 1), lambda i: (i, 0))
    return pl.pallas_call(
        body,
        out_shape=[
            jax.ShapeDtypeStruct((N, D), _f32),
            jax.ShapeDtypeStruct((N, 1), _f32),
            jax.ShapeDtypeStruct((N, 1), _f32),
        ],
        grid=(N // _BLK,),
        in_specs=[pl.BlockSpec((_BLK, D), lambda i: (i, 0))] + part + part,
        out_specs=[
            pl.BlockSpec((_BLK, D), lambda i: (i, 0)),
            col, col,
        ],
    )(h, deg_p, deg_p, cnt_p, cnt_p)


def _tc_combine_scale(a_p, dinv):
    def body(a0_ref, a1_ref, dinv_ref, o_ref):
        o_ref[...] = ((a0_ref[...].reshape(_BLK, D)
                       + a1_ref[...].reshape(_BLK, D)) * dinv_ref[...])
    return pl.pallas_call(
        body,
        out_shape=jax.ShapeDtypeStruct((N, D), _f32),
        grid=(N // _BLK,),
        in_specs=[
            pl.BlockSpec((1, _BLK, D), _p0),
            pl.BlockSpec((1, _BLK, D), _p1),
            pl.BlockSpec((_BLK, 1), lambda i: (i, 0)),
        ],
        out_specs=pl.BlockSpec((_BLK, D), lambda i: (i, 0)),
    )(a_p, a_p, dinv)


def _tc_root(h_gcn, W_r, b_row):
    def body(hg_ref, wr_ref, bs_ref, o_ref):
        o_ref[...] = (jnp.dot(hg_ref[...], wr_ref[...],
                              preferred_element_type=_f32) + bs_ref[...])
    return pl.pallas_call(
        body,
        out_shape=jax.ShapeDtypeStruct((N, D), _f32),
        grid=(N // _BLK,),
        in_specs=[
            pl.BlockSpec((_BLK, D), lambda i: (i, 0)),
            pl.BlockSpec((D, D), lambda i: (0, 0)),
            pl.BlockSpec((1, D), lambda i: (0, 0)),
        ],
        out_specs=pl.BlockSpec((_BLK, D), lambda i: (i, 0)),
    )(h_gcn, W_r, b_row)


def _tc_final(b_p, invc, base, W_l):
    def body(b0_ref, b1_ref, ic_ref, base_ref, wl_ref, o_ref):
        mean = (b0_ref[...].reshape(_BLK, D)
                + b1_ref[...].reshape(_BLK, D)) * ic_ref[...]
        acc = (jnp.dot(mean, wl_ref[...], preferred_element_type=_f32)
               + base_ref[...])
        ss = jnp.sum(acc * acc, axis=1, keepdims=True)
        o_ref[...] = acc / jnp.clip(jnp.sqrt(ss), 1e-12, None)
    return pl.pallas_call(
        body,
        out_shape=jax.ShapeDtypeStruct((N, D), _f32),
        grid=(N // _BLK,),
        in_specs=[
            pl.BlockSpec((1, _BLK, D), _p0),
            pl.BlockSpec((1, _BLK, D), _p1),
            pl.BlockSpec((_BLK, 1), lambda i: (i, 0)),
            pl.BlockSpec((_BLK, D), lambda i: (i, 0)),
            pl.BlockSpec((D, D), lambda i: (0, 0)),
        ],
        out_specs=pl.BlockSpec((_BLK, D), lambda i: (i, 0)),
    )(b_p, b_p, invc, base, W_l)


_SC_DEG_CNT = _make_sc_deg_cnt()
_SC_AGG_SCALED = _make_sc_agg(scaled=True)
_SC_AGG_PLAIN = _make_sc_agg(scaled=False)


def kernel(x, edge_index, attr, W_gcn, b_gcn, W_l, W_r, b_sage):
    E = edge_index.shape[1]
    pad = EPAD - E
    src = edge_index[0].astype(_i32)
    dst = edge_index[1].astype(_i32)
    w = attr.reshape(-1).astype(_f32)
    ar = jnp.arange(pad, dtype=_i32)
    pk_pad = (ar % N) | ((N + ar % (NPAD - N)) << 16)
    pk_p = jnp.concatenate([src | (dst << 16), pk_pad])
    w_p = jnp.concatenate([w, jnp.zeros((pad,), _f32)])
    w2d = w_p.reshape(EPAD // WIN, WIN)
    pk2d = pk_p.reshape(EPAD // WIN, WIN)

    deg_p, cnt_p = _SC_DEG_CNT(pk2d, w2d)
    h = _tc_matmul_bias(x, W_gcn, b_gcn.reshape(1, D))
    htil, dinv, invc = _tc_norm1(h, deg_p.reshape(2, NPAD, 1),
                                 cnt_p.reshape(2, NPAD, 1))
    a_p = _SC_AGG_SCALED(pk2d, w2d, htil)
    h_gcn = _tc_combine_scale(a_p, dinv)
    b_p = _SC_AGG_PLAIN(pk2d, h_gcn)
    base = _tc_root(h_gcn, W_r, b_sage.reshape(1, D))
    return _tc_final(b_p, invc, base, W_l)


# speedup vs baseline: 1.1021x; 1.1021x over previous
import functools

import jax
import jax.numpy as jnp
from jax import lax
from jax.experimental import pallas as pl
from jax.experimental.pallas import tpu as pltpu
from jax.experimental.pallas import tpu_sc as plsc

N = 10000
D = 128
NPAD = 10240
EPAD = 327680
NW = 32
EPT = EPAD // NW
WIN = 128
NWIN = EPT // WIN
RPT = NPAD // 16

_f32 = jnp.float32
_i32 = jnp.int32


def _mesh():
    return plsc.VectorSubcoreMesh(core_axis_name="c", subcore_axis_name="s",
                                  num_cores=2, num_subcores=16)


def _make_sc_deg_cnt():
    @functools.partial(
        pl.kernel,
        out_type=[
            jax.ShapeDtypeStruct((2, NPAD), _f32),
            jax.ShapeDtypeStruct((2, NPAD), _f32),
        ],
        mesh=_mesh(),
        scratch_types=[
            pltpu.VMEM((NWIN, WIN), _i32),
            pltpu.VMEM((NWIN, WIN), _f32),
            pltpu.VMEM((4, WIN), _i32),
            pltpu.VMEM((WIN,), _f32),
            pltpu.VMEM((RPT,), _f32),
            pltpu.VMEM_SHARED((NPAD,), _f32),
            pltpu.VMEM_SHARED((NPAD,), _f32),
        ] + [pltpu.SemaphoreType.DMA] * 8,
        compiler_params=pltpu.CompilerParams(needs_layout_passes=False),
    )
    def sc_deg_cnt(pk_hbm, w_hbm, deg_out, cnt_out,
                   pkv, wv, didx, onesv, bncv, deg_s, cnt_s, *sems):
        sems_a, sems_b = sems[0:4], sems[4:8]
        c = lax.axis_index("c")
        s = lax.axis_index("s")
        wid = c * 16 + s
        pltpu.sync_copy(pk_hbm.at[pl.ds(wid * NWIN, NWIN), :], pkv)
        pltpu.sync_copy(w_hbm.at[pl.ds(wid * NWIN, NWIN), :], wv)
        zeros16 = jnp.zeros((16,), _f32)
        ones16 = jnp.ones((16,), _f32)
        for k in range(WIN // 16):
            onesv[pl.ds(k * 16, 16)] = ones16

        def zbody(i, carry):
            bncv[pl.ds(i * 16, 16)] = zeros16
            return carry
        lax.fori_loop(0, RPT // 16, zbody, 0)
        pltpu.sync_copy(bncv, deg_s.at[pl.ds(s * RPT, RPT)])
        pltpu.sync_copy(bncv, cnt_s.at[pl.ds(s * RPT, RPT)])
        plsc.subcore_barrier()

        def issue(j, r):
            for k in range(WIN // 16):
                v = pkv[j, pl.ds(k * 16, 16)]
                didx[r, pl.ds(k * 16, 16)] = v >> 16
            pltpu.async_copy(wv.at[j], deg_s.at[didx.at[r]], sems_a[r],
                             add=True)
            pltpu.async_copy(onesv, cnt_s.at[didx.at[r]], sems_b[r],
                             add=True)

        def drain(r):
            pltpu.make_async_copy(wv.at[0], deg_s.at[didx.at[r]],
                                  sems_a[r]).wait()
            pltpu.make_async_copy(onesv, cnt_s.at[didx.at[r]],
                                  sems_b[r]).wait()

        for r in range(4):
            issue(jnp.int32(r), r)

        def body(q, carry):
            for r in range(4):
                drain(r)
                issue(4 * q + r, r)
            return carry
        lax.fori_loop(1, NWIN // 4, body, 0)
        for r in range(4):
            drain(r)
        plsc.subcore_barrier()
        pltpu.sync_copy(deg_s.at[pl.ds(s * RPT, RPT)], bncv)
        pltpu.sync_copy(bncv, deg_out.at[c, pl.ds(s * RPT, RPT)])
        pltpu.sync_copy(cnt_s.at[pl.ds(s * RPT, RPT)], bncv)
        pltpu.sync_copy(bncv, cnt_out.at[c, pl.ds(s * RPT, RPT)])

    return sc_deg_cnt


_WIN = 64
_NWINS = EPT // _WIN
_NROW = EPT // 128
_NBUF = 4


def _make_sc_agg(scaled: bool):
    scratch = [
        pltpu.VMEM((_NROW, 128), _i32),
        pltpu.VMEM((_NBUF, 128), _i32),
        pltpu.VMEM((_NBUF, 128), _i32),
        pltpu.VMEM((_WIN, D), _f32),
        pltpu.VMEM((_WIN, D), _f32),
        pltpu.VMEM((_WIN, D), _f32),
        pltpu.VMEM((_WIN, D), _f32),
        pltpu.VMEM_SHARED((NPAD, D), _f32),
    ] + [pltpu.SemaphoreType.DMA] * (3 * _NBUF if scaled else 2 * _NBUF)
    if scaled:
        scratch.insert(3, pltpu.VMEM((_NBUF, 128), _f32))

    @functools.partial(
        pl.kernel,
        out_type=jax.ShapeDtypeStruct((2, NPAD, D), _f32),
        mesh=_mesh(),
        scratch_types=scratch,
        compiler_params=pltpu.CompilerParams(needs_layout_passes=False),
    )
    def sc_agg(*refs):
        if scaled:
            (pk_hbm, w_hbm, rows_hbm, out_hbm,
             pkv, sidxv, didxv, wwin, gb0, gb1, gb2, gb3, acc_s,
             *sems) = refs
            gsems, ssems, wsems = sems[0:4], sems[4:8], sems[8:12]
        else:
            (pk_hbm, rows_hbm, out_hbm,
             pkv, sidxv, didxv, gb0, gb1, gb2, gb3, acc_s, *sems) = refs
            gsems, ssems = sems[0:4], sems[4:8]
        gbs = [gb0, gb1, gb2, gb3]
        c = lax.axis_index("c")
        s = lax.axis_index("s")
        wid = c * 16 + s
        pltpu.sync_copy(pk_hbm.at[pl.ds(wid * _NROW, _NROW), :], pkv)
        zeros16 = jnp.zeros((16,), _f32)

        def zrow(i, carry):
            for k in range(D // 16):
                gb0[i, pl.ds(k * 16, 16)] = zeros16
            return carry
        lax.fori_loop(0, _WIN, zrow, 0)
        for b in range(RPT // _WIN):
            pltpu.sync_copy(gb0, acc_s.at[pl.ds(s * RPT + b * _WIN, _WIN), :])
        plsc.subcore_barrier()

        mask16 = jnp.full((16,), 0xFFFF, _i32)

        def unpack(wj, h, slot):
            row = wj // 2
            for k in range(_WIN // 16):
                v = pkv[row, pl.ds(h * _WIN + k * 16, 16)]
                sidxv[slot, pl.ds(k * 16, 16)] = v & mask16
                didxv[slot, pl.ds(k * 16, 16)] = v >> 16

        def gather(slot, gsem):
            pltpu.async_copy(
                rows_hbm.at[sidxv.at[slot, pl.ds(0, _WIN)]], gbs[slot],
                gsems[slot] if gsem is None else gsem)

        def gwait(slot):
            pltpu.make_async_copy(
                rows_hbm.at[sidxv.at[slot, pl.ds(0, _WIN)]], gbs[slot],
                gsems[slot]).wait()

        def scatter(slot):
            pltpu.async_copy(gbs[slot],
                             acc_s.at[didxv.at[slot, pl.ds(0, _WIN)]],
                             ssems[slot], add=True)

        def swait(slot):
            pltpu.make_async_copy(gbs[slot],
                                  acc_s.at[didxv.at[slot, pl.ds(0, _WIN)]],
                                  ssems[slot]).wait()

        def wcopy(wj, h, slot):
            pltpu.async_copy(w_hbm.at[wid * _NROW + wj // 2,
                                      pl.ds(h * _WIN, _WIN)],
                             wwin.at[slot, pl.ds(0, _WIN)], wsems[slot])

        def wwait(slot):
            pltpu.make_async_copy(w_hbm.at[wid * _NROW,
                                           pl.ds(0, _WIN)],
                                  wwin.at[slot, pl.ds(0, _WIN)],
                                  wsems[slot]).wait()

        def scale(slot):
            gb = gbs[slot]
            for kk in range(_WIN // 16):
                w16 = wwin[slot, pl.ds(kk * 16, 16)]
                for r in range(16):
                    i = kk * 16 + r
                    sc16 = jnp.broadcast_to(w16[r], (16,))
                    for k in range(D // 16):
                        gb[i, pl.ds(k * 16, 16)] = (
                            gb[i, pl.ds(k * 16, 16)] * sc16)

        def step(j, r, first, wsync):
            gwait(r)
            if scaled:
                if not wsync:
                    wwait(r)
                scale(r)
            scatter(r)
            pm = (r + 3) % _NBUF
            if not first:
                swait(pm)
            nj = lax.rem(j + 3, _NWINS)
            nh = (r + 3) % 2
            unpack(nj, nh, pm)
            if scaled:
                wcopy(nj, nh, pm)
            gather(pm, None)

        for wj in range(3):
            unpack(jnp.int32(wj), wj % 2, wj)
            if scaled:
                pltpu.sync_copy(
                    w_hbm.at[wid * _NROW + wj // 2,
                             pl.ds((wj % 2) * _WIN, _WIN)],
                    wwin.at[wj, pl.ds(0, _WIN)])
            gather(wj, None)
        step(jnp.int32(0), 0, True, True)
        step(jnp.int32(1), 1, False, True)
        step(jnp.int32(2), 2, False, True)
        step(jnp.int32(3), 3, False, False)

        def body(q, carry):
            j = 4 * q
            step(j, 0, False, False)
            step(j + 1, 1, False, False)
            step(j + 2, 2, False, False)
            step(j + 3, 3, False, False)
            return carry
        lax.fori_loop(1, _NWINS // 4, body, 0)
        swait(3)
        for slot in range(3):
            gwait(slot)
            if scaled:
                wwait(slot)

        plsc.subcore_barrier()
        for b in range(RPT // _WIN):
            pltpu.sync_copy(acc_s.at[pl.ds(s * RPT + b * _WIN, _WIN), :], gb0)
            pltpu.sync_copy(gb0, out_hbm.at[c, pl.ds(s * RPT + b * _WIN, _WIN), :])

    return sc_agg


_BLK = 1000


def _p0(i):
    return (0, i, 0)


def _p1(i):
    return (1, i, 0)


def _tc_matmul_bias(x, W, b_row):
    def body(x_ref, w_ref, b_ref, o_ref):
        o_ref[...] = (jnp.dot(x_ref[...], w_ref[...],
                              preferred_element_type=_f32) + b_ref[...])
    return pl.pallas_call(
        body,
        out_shape=jax.ShapeDtypeStruct((N, D), _f32),
        grid=(N // _BLK,),
        in_specs=[
            pl.BlockSpec((_BLK, D), lambda i: (i, 0)),
            pl.BlockSpec((D, D), lambda i: (0, 0)),
            pl.BlockSpec((1, D), lambda i: (0, 0)),
        ],
        out_specs=pl.BlockSpec((_BLK, D), lambda i: (i, 0)),
    )(x, W, b_row)


def _tc_norm1(h, deg_p, cnt_p):
    def body(h_ref, d0, d1, c0, c1, ht_ref, dinv_ref, invc_ref):
        deg = d0[...].reshape(_BLK, 1) + d1[...].reshape(_BLK, 1)
        dinv = lax.rsqrt(jnp.clip(deg, 1e-12, None))
        cnt = jnp.clip(c0[...].reshape(_BLK, 1) + c1[...].reshape(_BLK, 1),
                       1.0, None)
        dinv_ref[...] = dinv
        invc_ref[...] = 1.0 / cnt
        ht_ref[...] = h_ref[...] * dinv
    part = [pl.BlockSpec((1, _BLK, 1), _p0), pl.BlockSpec((1, _BLK, 1), _p1)]
    col = pl.BlockSpec((_BLK, 1), lambda i: (i, 0))
    return pl.pallas_call(
        body,
        out_shape=[
            jax.ShapeDtypeStruct((N, D), _f32),
            jax.ShapeDtypeStruct((N, 1), _f32),
            jax.ShapeDtypeStruct((N, 1), _f32),
        ],
        grid=(N // _BLK,),
        in_specs=[pl.BlockSpec((_BLK, D), lambda i: (i, 0))] + part + part,
        out_specs=[
            pl.BlockSpec((_BLK, D), lambda i: (i, 0)),
            col, col,
        ],
    )(h, deg_p, deg_p, cnt_p, cnt_p)


def _tc_combine_scale(a_p, dinv):
    def body(a0_ref, a1_ref, dinv_ref, o_ref):
        o_ref[...] = ((a0_ref[...].reshape(_BLK, D)
                       + a1_ref[...].reshape(_BLK, D)) * dinv_ref[...])
    return pl.pallas_call(
        body,
        out_shape=jax.ShapeDtypeStruct((N, D), _f32),
        grid=(N // _BLK,),
        in_specs=[
            pl.BlockSpec((1, _BLK, D), _p0),
            pl.BlockSpec((1, _BLK, D), _p1),
            pl.BlockSpec((_BLK, 1), lambda i: (i, 0)),
        ],
        out_specs=pl.BlockSpec((_BLK, D), lambda i: (i, 0)),
    )(a_p, a_p, dinv)


def _tc_root(h_gcn, W_r, b_row):
    def body(hg_ref, wr_ref, bs_ref, o_ref):
        o_ref[...] = (jnp.dot(hg_ref[...], wr_ref[...],
                              preferred_element_type=_f32) + bs_ref[...])
    return pl.pallas_call(
        body,
        out_shape=jax.ShapeDtypeStruct((N, D), _f32),
        grid=(N // _BLK,),
        in_specs=[
            pl.BlockSpec((_BLK, D), lambda i: (i, 0)),
            pl.BlockSpec((D, D), lambda i: (0, 0)),
            pl.BlockSpec((1, D), lambda i: (0, 0)),
        ],
        out_specs=pl.BlockSpec((_BLK, D), lambda i: (i, 0)),
    )(h_gcn, W_r, b_row)


def _tc_final(b_p, invc, base, W_l):
    def body(b0_ref, b1_ref, ic_ref, base_ref, wl_ref, o_ref):
        mean = (b0_ref[...].reshape(_BLK, D)
                + b1_ref[...].reshape(_BLK, D)) * ic_ref[...]
        acc = (jnp.dot(mean, wl_ref[...], preferred_element_type=_f32)
               + base_ref[...])
        ss = jnp.sum(acc * acc, axis=1, keepdims=True)
        o_ref[...] = acc / jnp.clip(jnp.sqrt(ss), 1e-12, None)
    return pl.pallas_call(
        body,
        out_shape=jax.ShapeDtypeStruct((N, D), _f32),
        grid=(N // _BLK,),
        in_specs=[
            pl.BlockSpec((1, _BLK, D), _p0),
            pl.BlockSpec((1, _BLK, D), _p1),
            pl.BlockSpec((_BLK, 1), lambda i: (i, 0)),
            pl.BlockSpec((_BLK, D), lambda i: (i, 0)),
            pl.BlockSpec((D, D), lambda i: (0, 0)),
        ],
        out_specs=pl.BlockSpec((_BLK, D), lambda i: (i, 0)),
    )(b_p, b_p, invc, base, W_l)


_SC_DEG_CNT = _make_sc_deg_cnt()
_SC_AGG_SCALED = _make_sc_agg(scaled=True)
_SC_AGG_PLAIN = _make_sc_agg(scaled=False)


def kernel(x, edge_index, attr, W_gcn, b_gcn, W_l, W_r, b_sage):
    E = edge_index.shape[1]
    pad = EPAD - E
    src = edge_index[0].astype(_i32)
    dst = edge_index[1].astype(_i32)
    w = attr.reshape(-1).astype(_f32)
    ar = jnp.arange(pad, dtype=_i32)
    pk_pad = (ar % N) | ((N + ar % (NPAD - N)) << 16)
    pk_p = jnp.concatenate([src | (dst << 16), pk_pad])
    w_p = jnp.concatenate([w, jnp.zeros((pad,), _f32)])
    w2d = w_p.reshape(EPAD // WIN, WIN)
    pk2d = pk_p.reshape(EPAD // WIN, WIN)

    deg_p, cnt_p = _SC_DEG_CNT(pk2d, w2d)
    h = _tc_matmul_bias(x, W_gcn, b_gcn.reshape(1, D))
    htil, dinv, invc = _tc_norm1(h, deg_p.reshape(2, NPAD, 1),
                                 cnt_p.reshape(2, NPAD, 1))
    a_p = _SC_AGG_SCALED(pk2d, w2d, htil)
    h_gcn = _tc_combine_scale(a_p, dinv)
    b_p = _SC_AGG_PLAIN(pk2d, h_gcn)
    base = _tc_root(h_gcn, W_r, b_sage.reshape(1, D))
    return _tc_final(b_p, invc, base, W_l)
